# Initial kernel scaffold; baseline (speedup 1.0000x reference)
import jax
import jax.numpy as jnp
from jax.experimental import pallas as pl


def kernel(emb_e, conv1_basis, conv1_att, conv1_root, conv1_bias, conv2_basis, conv2_att, conv2_root, conv2_bias, entity, edge_index, edge_type, edge_norm):
    raise NotImplementedError("write your pallas kernel here")


# speedup vs baseline: 1.0185x; 1.0185x over previous
import functools

import jax
import jax.numpy as jnp
from jax.experimental import pallas as pl
from jax.experimental.pallas import tpu as pltpu


def _round_up(a, m):
    return ((a + m - 1) // m) * m


def _layer_kernel(first_ref, ntiles_ref, wlo_ref, whi_ref,
                  dst_ref, xj_ref, coeff_ref, w_ref,
                  x_ref, root_ref, bias_ref,
                  out_ref,
                  *, num_bases, d_out, half, win, t_e, apply_relu):
    c = pl.program_id(0)
    k = pl.program_id(1)

    @pl.when(k == 0)
    def _init():
        out_ref[...] = jnp.zeros_like(out_ref)

    @pl.when(k < ntiles_ref[c])
    def _accumulate():
        t = first_ref[c] + k
        y = jnp.dot(xj_ref[...], w_ref[...],
                    preferred_element_type=jnp.float32)
        coeff = coeff_ref[...]
        msgs = coeff[:, 0:1] * y[:, 0:d_out]
        for b in range(1, num_bases):
            msgs = msgs + coeff[:, b:b + 1] * y[:, b * d_out:(b + 1) * d_out]
        msgs = msgs.astype(jnp.bfloat16)
        dstv = dst_ref[...]

        wpc = half // win
        w_lo = jnp.maximum(wlo_ref[t], c * wpc)
        w_hi = jnp.minimum(whi_ref[t], c * wpc + wpc - 1)

        def _scatter(j, carry):
            w0 = w_lo + j
            row0 = w0 * win
            rows = jax.lax.broadcasted_iota(jnp.int32, (win, t_e), 0) + row0
            oh = (rows == dstv).astype(jnp.bfloat16)
            part = jnp.dot(oh, msgs, preferred_element_type=jnp.float32)
            out_ref[pl.ds(row0 - c * half, win), :] += part
            return carry

        jax.lax.fori_loop(0, jnp.maximum(w_hi - w_lo + 1, 0), _scatter, 0)

    @pl.when(k == pl.num_programs(1) - 1)
    def _finalize():
        res = out_ref[...] + jnp.dot(x_ref[...], root_ref[...],
                                     preferred_element_type=jnp.float32)
        res = res + bias_ref[...]
        if apply_relu:
            res = jnp.maximum(res, 0.0)
        out_ref[...] = res


def _tile_idx(c, k, first, ntiles, wlo, whi):
    return first[c] + jnp.minimum(k, jnp.maximum(ntiles[c] - 1, 0))


def _dst_map(c, k, first, ntiles, wlo, whi):
    return (0, _tile_idx(c, k, first, ntiles, wlo, whi))


def _edge_map(c, k, first, ntiles, wlo, whi):
    return (_tile_idx(c, k, first, ntiles, wlo, whi), 0)


def _rgcn_layer(x, meta, att, basis, root, bias, *, apply_relu, t_e, win):
    (src_pad, dst_row, type_pad, scale_pad,
     first_tile, ntiles, wlo, whi, kb, half) = meta
    n_pad, d_in = x.shape
    num_bases, _, d_out = basis.shape
    e_pad = src_pad.shape[0]

    x_bf = x.astype(jnp.bfloat16)
    xj = jnp.take(x_bf, src_pad, axis=0)
    coeff = jnp.take(att.astype(jnp.float32), type_pad, axis=0) \
        * scale_pad[:, None]
    w_stacked = jnp.transpose(basis, (1, 0, 2)).reshape(
        d_in, num_bases * d_out).astype(jnp.bfloat16)
    root_bf = root.astype(jnp.bfloat16)
    bias2 = bias.reshape(1, d_out).astype(jnp.float32)

    kernel_fn = functools.partial(
        _layer_kernel, num_bases=num_bases, d_out=d_out,
        half=half, win=win, t_e=t_e, apply_relu=apply_relu)

    lane = lambda c_: _round_up(c_, 128)
    vmem_bytes = (
        2 * (2 * t_e * lane(d_in)
             + 4 * t_e * lane(num_bases)
             + 4 * 8 * t_e)
        + 4 * t_e * lane(num_bases * d_out)
        + 2 * t_e * lane(d_out)
        + 2 * win * t_e
        + 4 * half * lane(d_out)
        + 2 * half * lane(d_in)
        + 2 * d_in * lane(num_bases * d_out)
        + 2 * d_in * lane(d_out) + 4 * 8 * lane(d_out))
    vmem_limit = int(min(max(2 * vmem_bytes, 4 << 20), 48 << 20))

    cost = pl.CostEstimate(
        flops=int(2 * e_pad * d_in * num_bases * d_out
                  + 2 * e_pad * 2 * win * d_out
                  + 2 * n_pad * d_in * d_out),
        transcendentals=0,
        bytes_accessed=int(2 * e_pad * (d_in + 2 * num_bases + 2)
                           + 4 * n_pad * (d_in + d_out)
                           + 2 * d_in * (num_bases * d_out + d_out)),
    )

    out = pl.pallas_call(
        kernel_fn,
        out_shape=jax.ShapeDtypeStruct((n_pad, d_out), jnp.float32),
        grid_spec=pltpu.PrefetchScalarGridSpec(
            num_scalar_prefetch=4,
            grid=(2, kb),
            in_specs=[
                pl.BlockSpec((1, t_e), _dst_map),
                pl.BlockSpec((t_e, d_in), _edge_map),
                pl.BlockSpec((t_e, num_bases), _edge_map),
                pl.BlockSpec((d_in, num_bases * d_out),
                             lambda c, k, *_: (0, 0)),
                pl.BlockSpec((half, d_in), lambda c, k, *_: (c, 0)),
                pl.BlockSpec((d_in, d_out), lambda c, k, *_: (0, 0)),
                pl.BlockSpec((1, d_out), lambda c, k, *_: (0, 0)),
            ],
            out_specs=pl.BlockSpec((half, d_out), lambda c, k, *_: (c, 0)),
        ),
        compiler_params=pltpu.CompilerParams(
            dimension_semantics=("parallel", "arbitrary"),
            vmem_limit_bytes=vmem_limit,
        ),
        cost_estimate=cost,
    )(first_tile, ntiles, wlo, whi,
      dst_row, xj, coeff, w_stacked, x_bf, root_bf, bias2)
    return out


def _edge_meta(edge_index, edge_type, edge_norm, n_nodes, *, t_e, win):
    src = edge_index[0].astype(jnp.int32)
    dst = edge_index[1].astype(jnp.int32)
    n_edges = dst.shape[0]
    n_pad = _round_up(n_nodes, 2 * win)
    half = n_pad // 2

    order = jnp.argsort(dst)
    src_s = src[order]
    dst_s = dst[order]
    type_s = edge_type.astype(jnp.int32)[order]
    norm_s = edge_norm.astype(jnp.float32)[order]

    bounds = jnp.searchsorted(dst_s, jnp.arange(n_nodes + 1, dtype=jnp.int32))
    deg = jnp.maximum((bounds[1:] - bounds[:-1]).astype(jnp.float32), 1.0)
    scale = norm_s / deg[dst_s]

    e_pad = _round_up(max(n_edges, 1), t_e)
    pad_e = e_pad - n_edges
    src_pad = jnp.pad(src_s, (0, pad_e))
    dst_pad = jnp.pad(dst_s, (0, pad_e), constant_values=n_pad)
    type_pad = jnp.pad(type_s, (0, pad_e))
    scale_pad = jnp.pad(scale, (0, pad_e))
    kb = e_pad // t_e

    wlo = (dst_pad[0::t_e] // win).astype(jnp.int32)
    whi = (dst_pad[t_e - 1::t_e] // win).astype(jnp.int32)

    cuts = jnp.array([0, half], jnp.int32)
    lo = jnp.searchsorted(dst_s, cuts, side='left').astype(jnp.int32)
    hi = jnp.searchsorted(dst_s, cuts + half, side='left').astype(jnp.int32)
    empty = hi <= lo
    first_tile = jnp.where(empty, 0, lo // t_e).astype(jnp.int32)
    ntiles = jnp.where(
        empty, 0, jnp.maximum(hi - 1, 0) // t_e - lo // t_e + 1).astype(jnp.int32)

    dst_row = dst_pad.reshape(1, e_pad)
    return (src_pad, dst_row, type_pad, scale_pad,
            first_tile, ntiles, wlo, whi, kb, half), n_pad


def kernel(emb_e, conv1_basis, conv1_att, conv1_root, conv1_bias,
           conv2_basis, conv2_att, conv2_root, conv2_bias,
           entity, edge_index, edge_type, edge_norm,
           *, edge_tile=2048, node_win=256):
    n_nodes = entity.shape[0]
    x = jnp.take(emb_e, jnp.squeeze(entity), axis=0).astype(jnp.float32)

    meta, n_pad = _edge_meta(edge_index, edge_type, edge_norm, n_nodes,
                             t_e=edge_tile, win=node_win)
    x = jnp.pad(x, ((0, n_pad - n_nodes), (0, 0)))

    x = _rgcn_layer(x, meta, conv1_att, conv1_basis, conv1_root, conv1_bias,
                    apply_relu=True, t_e=edge_tile, win=node_win)
    x = _rgcn_layer(x, meta, conv2_att, conv2_basis, conv2_root, conv2_bias,
                    apply_relu=False, t_e=edge_tile, win=node_win)
    return x[:n_nodes]
